# baseline (speedup 1.0000x reference)
import jax
import jax.numpy as jnp
from jax.experimental import pallas as pl
from jax.experimental.pallas import tpu as pltpu

_C = 17
_DIM = 64
_NVIEW = 100
_QSIZE = 150
_UPD = 30
_MU = 0.99
_TEMP = 0.1
_B = 8
_N = 16384
_BC = _B * _C
_NQ = _C * _QSIZE
_NCT = _NQ + _C
_CTP = 2688
_BIG = 1 << 30


def _topk_gather_kernel(yh_ref, y_ref, tie_ref, feats_ref, x_ref, val_ref,
                        score_ref):
    c = pl.program_id(1)
    yh = yh_ref[0]
    yy = y_ref[0]
    tie = tie_ref[0]
    mask = (yh == c)
    hard = jnp.logical_and(mask, yy != c)
    score_ref[...] = (mask.astype(jnp.float32) * 4.0
                      + hard.astype(jnp.float32) + tie)
    x_ref[0] = jnp.zeros((128, _DIM), jnp.float32)
    lin = jax.lax.broadcasted_iota(jnp.int32, (1, _N), 1)
    lane = jax.lax.broadcasted_iota(jnp.int32, (1, 128), 1)

    def body(k, valvec):
        s = score_ref[...]
        m = jnp.max(s)
        sel = jnp.min(jnp.where(s == m, lin, _BIG))
        row = feats_ref[0, pl.ds(sel, 1), :]
        x_ref[0, pl.ds(k, 1), :] = row
        score_ref[...] = jnp.where(lin == sel, -1.0, s)
        vk = (m >= 4.0).astype(jnp.float32)
        return jnp.where(lane == k, vk, valvec)

    valvec = jax.lax.fori_loop(0, _NVIEW, body, jnp.zeros((1, 128),
                                                          jnp.float32))
    val_ref[0] = valvec


def _loss_kernel(x_ref, val_ref, ct_ref, num_ref, den_ref, loss_ref):
    step = pl.program_id(0)
    c = jnp.mod(step, _C)
    x = x_ref[0]
    nrm = jnp.sqrt(jnp.sum(x * x, axis=1, keepdims=True))
    a = x / jnp.maximum(nrm, 1e-12)
    logits = jax.lax.dot_general(a, ct_ref[...], (((1,), (0,)), ((), ())),
                                 preferred_element_type=jnp.float32) / _TEMP
    col = jax.lax.broadcasted_iota(jnp.int32, (128, _CTP), 1)
    clab = jnp.where(col < _NQ, col // _QSIZE,
                     jnp.where(col < _NCT, col - _NQ, -1))
    real = col < _NCT
    lm = jnp.where(real, logits, -1e30)
    m = jnp.max(lm, axis=1, keepdims=True)
    ex = jnp.where(real, jnp.exp(lm - m), 0.0)
    lse = jnp.log(jnp.sum(ex, axis=1, keepdims=True))
    pos = (clab == c)
    spos = jnp.sum(jnp.where(pos, lm - m - lse, 0.0), axis=1, keepdims=True)
    per = -_TEMP * (spos / (_QSIZE + 1.0))
    v = val_ref[0]
    nm = jax.lax.dot_general(v, per, (((1,), (0,)), ((), ())),
                             preferred_element_type=jnp.float32)
    dn = jnp.sum(v).reshape(1, 1)

    @pl.when(step == 0)
    def _():
        num_ref[...] = nm
        den_ref[...] = dn
        loss_ref[...] = jnp.zeros((1, 1), jnp.float32)

    @pl.when(step > 0)
    def _():
        num_ref[...] = num_ref[...] + nm
        den_ref[...] = den_ref[...] + dn

    @pl.when(step == _BC - 1)
    def _():
        loss_ref[...] = num_ref[...] / jnp.maximum(den_ref[...], 1.0)


def _queue_kernel(x_ref, val_ref, tie2_ref, q_ref, cen_ref, ptr_ref,
                  qout_ref, cout_ref):
    p0 = ptr_ref[0, 0, 0]
    val = val_ref[:, 0, 0, :]
    sv0 = val * 2.0 + tie2_ref[0]
    lin = (jax.lax.broadcasted_iota(jnp.int32, (_B, 128), 0) * 128
           + jax.lax.broadcasted_iota(jnp.int32, (_B, 128), 1))
    qout_ref[0] = q_ref[0]

    def body(k, sv):
        m = jnp.max(sv)
        sel = jnp.min(jnp.where(sv == m, lin, _BIG))
        bs = sel // 128
        js = jnp.mod(sel, 128)
        row = x_ref[pl.ds(bs, 1), 0, pl.ds(js, 1), :].reshape(1, _DIM)
        nr = jnp.sqrt(jnp.sum(row * row))
        r = row / jnp.maximum(nr, 1e-12)
        p = jnp.mod(p0 + k, _QSIZE)
        qout_ref[0, pl.ds(p, 1), :] = r
        return jnp.where(lin == sel, -20.0, sv)

    jax.lax.fori_loop(0, _UPD, body, sv0)

    x = x_ref[:, 0, :, :]
    nrm = jnp.sqrt(jnp.sum(x * x, axis=2, keepdims=True))
    an = x / jnp.maximum(nrm, 1e-12)
    wsum = jnp.sum(an * val[:, :, None], axis=(0, 1)).reshape(1, _DIM)
    cnt = jnp.maximum(jnp.sum(val), 1.0)
    newc = wsum / cnt
    newc = newc / jnp.maximum(jnp.sqrt(jnp.sum(newc * newc)), 1e-12)
    cen = _MU * cen_ref[0] + (1.0 - _MU) * newc
    cout_ref[0] = cen / jnp.maximum(jnp.sqrt(jnp.sum(cen * cen)), 1e-12)


def kernel(feats, y_hat, y, point_queue, cluster_center, point_queue_ptr):
    tie = jax.random.uniform(jax.random.key(42), (_B, _C, _N))
    tie2 = jax.random.uniform(jax.random.key(7), (_C, _B * _NVIEW))
    tie2p = jnp.pad(tie2.reshape(_C, _B, _NVIEW), ((0, 0), (0, 0), (0, 28)),
                    constant_values=-10.0)

    f32 = jnp.float32
    x_sel, valid = pl.pallas_call(
        _topk_gather_kernel,
        grid=(_B, _C),
        in_specs=[
            pl.BlockSpec((1, 1, _N), lambda b, c: (b, 0, 0)),
            pl.BlockSpec((1, 1, _N), lambda b, c: (b, 0, 0)),
            pl.BlockSpec((1, 1, _N), lambda b, c: (b * _C + c, 0, 0)),
            pl.BlockSpec((1, _N, _DIM), lambda b, c: (b, 0, 0)),
        ],
        out_specs=[
            pl.BlockSpec((1, 128, _DIM), lambda b, c: (b * _C + c, 0, 0)),
            pl.BlockSpec((1, 1, 128), lambda b, c: (b * _C + c, 0, 0)),
        ],
        out_shape=[
            jax.ShapeDtypeStruct((_BC, 128, _DIM), f32),
            jax.ShapeDtypeStruct((_BC, 1, 128), f32),
        ],
        scratch_shapes=[pltpu.VMEM((1, _N), f32)],
    )(y_hat.reshape(_B, 1, _N), y.reshape(_B, 1, _N),
      tie.reshape(_BC, 1, _N), feats)

    ct = jnp.concatenate(
        [point_queue.reshape(_NQ, _DIM), cluster_center], axis=0)
    ctp = jnp.pad(ct, ((0, _CTP - _NCT), (0, 0))).T

    num, den, loss = pl.pallas_call(
        _loss_kernel,
        grid=(_BC,),
        in_specs=[
            pl.BlockSpec((1, 128, _DIM), lambda i: (i, 0, 0)),
            pl.BlockSpec((1, 1, 128), lambda i: (i, 0, 0)),
            pl.BlockSpec((_DIM, _CTP), lambda i: (0, 0)),
        ],
        out_specs=[
            pl.BlockSpec((1, 1), lambda i: (0, 0)),
            pl.BlockSpec((1, 1), lambda i: (0, 0)),
            pl.BlockSpec((1, 1), lambda i: (0, 0)),
        ],
        out_shape=[
            jax.ShapeDtypeStruct((1, 1), f32),
            jax.ShapeDtypeStruct((1, 1), f32),
            jax.ShapeDtypeStruct((1, 1), f32),
        ],
    )(x_sel, valid, ctp)

    new_queue, center = pl.pallas_call(
        _queue_kernel,
        grid=(_C,),
        in_specs=[
            pl.BlockSpec((_B, 1, 128, _DIM), lambda c: (0, c, 0, 0)),
            pl.BlockSpec((_B, 1, 1, 128), lambda c: (0, c, 0, 0)),
            pl.BlockSpec((1, _B, 128), lambda c: (c, 0, 0)),
            pl.BlockSpec((1, _QSIZE, _DIM), lambda c: (c, 0, 0)),
            pl.BlockSpec((1, 1, _DIM), lambda c: (c, 0, 0)),
            pl.BlockSpec((1, 1, 1), lambda c: (c, 0, 0)),
        ],
        out_specs=[
            pl.BlockSpec((1, _QSIZE, _DIM), lambda c: (c, 0, 0)),
            pl.BlockSpec((1, 1, _DIM), lambda c: (c, 0, 0)),
        ],
        out_shape=[
            jax.ShapeDtypeStruct((_C, _QSIZE, _DIM), f32),
            jax.ShapeDtypeStruct((_C, 1, _DIM), f32),
        ],
    )(x_sel.reshape(_B, _C, 128, _DIM), valid.reshape(_B, _C, 1, 128),
      tie2p, point_queue, cluster_center.reshape(_C, 1, _DIM),
      point_queue_ptr.astype(jnp.int32).reshape(_C, 1, 1))

    new_ptr = jnp.mod(point_queue_ptr.astype(jnp.int32) + _UPD, _QSIZE)
    return (loss[0, 0], new_queue, center.reshape(_C, _DIM), new_ptr)


# speedup vs baseline: 1.5663x; 1.5663x over previous
import jax
import jax.numpy as jnp
from jax.experimental import pallas as pl
from jax.experimental.pallas import tpu as pltpu

_C = 17
_DIM = 64
_NVIEW = 100
_QSIZE = 150
_UPD = 30
_MU = 0.99
_TEMP = 0.1
_B = 8
_N = 16384
_BC = _B * _C
_NQ = _C * _QSIZE
_NCT = _NQ + _C
_CTP = 2688
_BIG = 1 << 30


def _topk_gather_kernel(yh_ref, y_ref, tie_ref, feats_ref, x_ref, val_ref,
                        score_ref, m_ref, lidx_ref):
    f32 = jnp.float32
    yh = yh_ref[0]
    yy = y_ref[0]
    gi = jax.lax.broadcasted_iota(jnp.int32, (128, 128), 0)
    ti = jax.lax.broadcasted_iota(jnp.int32, (128, 128), 1)
    lin2 = gi * 128 + ti
    tio = jax.lax.broadcasted_iota(jnp.int32, (1, 128), 1)
    x_ref[0] = jnp.zeros((_C * 128, _DIM), f32)
    for c in range(_C):
        mask = (yh == c)
        sc = (mask.astype(f32) * 4.0
              + jnp.logical_and(mask, yy != c).astype(f32)
              + tie_ref[0, c])
        score_ref[c] = sc
        mg = jnp.max(sc, axis=1, keepdims=True)
        m_ref[:, pl.ds(c, 1)] = mg
        lidx_ref[:, pl.ds(c, 1)] = jnp.min(
            jnp.where(sc == mg, lin2, _BIG), axis=1, keepdims=True)

    def body(k, vrows):
        m_g = m_ref[...]
        m_c = jnp.max(m_g, axis=0, keepdims=True)
        sel_v = jnp.min(jnp.where(m_g == m_c, lidx_ref[...], _BIG),
                        axis=0, keepdims=True)
        out = []
        for c in range(_C):
            sel = sel_v[0, c]
            g_c = sel // 128
            t_c = jnp.mod(sel, 128)
            row = feats_ref[0, pl.ds(sel, 1), :]
            x_ref[0, pl.ds(k + c * 128, 1), :] = row
            srow = score_ref[c, pl.ds(g_c, 1), :]
            srow = jnp.where(tio == t_c, -1.0, srow)
            score_ref[c, pl.ds(g_c, 1), :] = srow
            mnew = jnp.max(srow)
            m_ref[pl.ds(g_c, 1), pl.ds(c, 1)] = mnew.reshape(1, 1)
            lnew = jnp.min(jnp.where(srow == mnew, g_c * 128 + tio, _BIG))
            lidx_ref[pl.ds(g_c, 1), pl.ds(c, 1)] = lnew.reshape(1, 1)
            vc = (m_c[0, c] >= 4.0).astype(f32)
            out.append(jnp.where(tio == k, vc, vrows[c]))
        return tuple(out)

    vrows = jax.lax.fori_loop(0, _NVIEW, body,
                              tuple(jnp.zeros((1, 128), f32)
                                    for _ in range(_C)))
    for c in range(_C):
        val_ref[0, pl.ds(c, 1), :] = vrows[c]


def _loss_kernel(x_ref, val_ref, ct_ref, num_ref, den_ref, loss_ref):
    step = pl.program_id(0)
    c = jnp.mod(step, _C)
    x = x_ref[0]
    nrm = jnp.sqrt(jnp.sum(x * x, axis=1, keepdims=True))
    a = x / jnp.maximum(nrm, 1e-12)
    logits = jax.lax.dot_general(a, ct_ref[...], (((1,), (0,)), ((), ())),
                                 preferred_element_type=jnp.float32) / _TEMP
    col = jax.lax.broadcasted_iota(jnp.int32, (128, _CTP), 1)
    clab = jnp.where(col < _NQ, col // _QSIZE,
                     jnp.where(col < _NCT, col - _NQ, -1))
    real = col < _NCT
    lm = jnp.where(real, logits, -1e30)
    m = jnp.max(lm, axis=1, keepdims=True)
    ex = jnp.where(real, jnp.exp(lm - m), 0.0)
    lse = jnp.log(jnp.sum(ex, axis=1, keepdims=True))
    pos = (clab == c)
    spos = jnp.sum(jnp.where(pos, lm - m - lse, 0.0), axis=1, keepdims=True)
    per = -_TEMP * (spos / (_QSIZE + 1.0))
    v = val_ref[0]
    nm = jax.lax.dot_general(v, per, (((1,), (0,)), ((), ())),
                             preferred_element_type=jnp.float32)
    dn = jnp.sum(v).reshape(1, 1)

    @pl.when(step == 0)
    def _():
        num_ref[...] = nm
        den_ref[...] = dn
        loss_ref[...] = jnp.zeros((1, 1), jnp.float32)

    @pl.when(step > 0)
    def _():
        num_ref[...] = num_ref[...] + nm
        den_ref[...] = den_ref[...] + dn

    @pl.when(step == _BC - 1)
    def _():
        loss_ref[...] = num_ref[...] / jnp.maximum(den_ref[...], 1.0)


def _queue_kernel(x_ref, val_ref, tie2_ref, q_ref, cen_ref, ptr_ref,
                  qout_ref, cout_ref):
    p0 = ptr_ref[0, 0, 0]
    val = val_ref[:, 0, 0, :]
    sv0 = val * 2.0 + tie2_ref[0]
    lin = (jax.lax.broadcasted_iota(jnp.int32, (_B, 128), 0) * 128
           + jax.lax.broadcasted_iota(jnp.int32, (_B, 128), 1))
    qout_ref[0] = q_ref[0]

    def body(k, sv):
        m = jnp.max(sv)
        sel = jnp.min(jnp.where(sv == m, lin, _BIG))
        bs = sel // 128
        js = jnp.mod(sel, 128)
        row = x_ref[pl.ds(bs, 1), 0, pl.ds(js, 1), :].reshape(1, _DIM)
        nr = jnp.sqrt(jnp.sum(row * row))
        r = row / jnp.maximum(nr, 1e-12)
        p = jnp.mod(p0 + k, _QSIZE)
        qout_ref[0, pl.ds(p, 1), :] = r
        return jnp.where(lin == sel, -20.0, sv)

    jax.lax.fori_loop(0, _UPD, body, sv0)

    x = x_ref[:, 0, :, :]
    nrm = jnp.sqrt(jnp.sum(x * x, axis=2, keepdims=True))
    an = x / jnp.maximum(nrm, 1e-12)
    wsum = jnp.sum(an * val[:, :, None], axis=(0, 1)).reshape(1, _DIM)
    cnt = jnp.maximum(jnp.sum(val), 1.0)
    newc = wsum / cnt
    newc = newc / jnp.maximum(jnp.sqrt(jnp.sum(newc * newc)), 1e-12)
    cen = _MU * cen_ref[0] + (1.0 - _MU) * newc
    cout_ref[0] = cen / jnp.maximum(jnp.sqrt(jnp.sum(cen * cen)), 1e-12)


def kernel(feats, y_hat, y, point_queue, cluster_center, point_queue_ptr):
    tie = jax.random.uniform(jax.random.key(42), (_B, _C, _N))
    tie2 = jax.random.uniform(jax.random.key(7), (_C, _B * _NVIEW))
    tie2p = jnp.pad(tie2.reshape(_C, _B, _NVIEW), ((0, 0), (0, 0), (0, 28)),
                    constant_values=-10.0)

    f32 = jnp.float32
    x_raw, valid_raw = pl.pallas_call(
        _topk_gather_kernel,
        grid=(_B,),
        in_specs=[
            pl.BlockSpec((1, 128, 128), lambda b: (b, 0, 0)),
            pl.BlockSpec((1, 128, 128), lambda b: (b, 0, 0)),
            pl.BlockSpec((1, _C, 128, 128), lambda b: (b, 0, 0, 0)),
            pl.BlockSpec((1, _N, _DIM), lambda b: (b, 0, 0)),
        ],
        out_specs=[
            pl.BlockSpec((1, _C * 128, _DIM), lambda b: (b, 0, 0)),
            pl.BlockSpec((1, _C, 128), lambda b: (b, 0, 0)),
        ],
        out_shape=[
            jax.ShapeDtypeStruct((_B, _C * 128, _DIM), f32),
            jax.ShapeDtypeStruct((_B, _C, 128), f32),
        ],
        scratch_shapes=[pltpu.VMEM((_C, 128, 128), f32),
                        pltpu.VMEM((128, 32), f32),
                        pltpu.VMEM((128, 32), jnp.int32)],
    )(y_hat.reshape(_B, 128, 128), y.reshape(_B, 128, 128),
      tie.reshape(_B, _C, 128, 128), feats)
    x_sel = x_raw.reshape(_BC, 128, _DIM)
    valid = valid_raw.reshape(_BC, 1, 128)

    ct = jnp.concatenate(
        [point_queue.reshape(_NQ, _DIM), cluster_center], axis=0)
    ctp = jnp.pad(ct, ((0, _CTP - _NCT), (0, 0))).T

    num, den, loss = pl.pallas_call(
        _loss_kernel,
        grid=(_BC,),
        in_specs=[
            pl.BlockSpec((1, 128, _DIM), lambda i: (i, 0, 0)),
            pl.BlockSpec((1, 1, 128), lambda i: (i, 0, 0)),
            pl.BlockSpec((_DIM, _CTP), lambda i: (0, 0)),
        ],
        out_specs=[
            pl.BlockSpec((1, 1), lambda i: (0, 0)),
            pl.BlockSpec((1, 1), lambda i: (0, 0)),
            pl.BlockSpec((1, 1), lambda i: (0, 0)),
        ],
        out_shape=[
            jax.ShapeDtypeStruct((1, 1), f32),
            jax.ShapeDtypeStruct((1, 1), f32),
            jax.ShapeDtypeStruct((1, 1), f32),
        ],
    )(x_sel, valid, ctp)

    new_queue, center = pl.pallas_call(
        _queue_kernel,
        grid=(_C,),
        in_specs=[
            pl.BlockSpec((_B, 1, 128, _DIM), lambda c: (0, c, 0, 0)),
            pl.BlockSpec((_B, 1, 1, 128), lambda c: (0, c, 0, 0)),
            pl.BlockSpec((1, _B, 128), lambda c: (c, 0, 0)),
            pl.BlockSpec((1, _QSIZE, _DIM), lambda c: (c, 0, 0)),
            pl.BlockSpec((1, 1, _DIM), lambda c: (c, 0, 0)),
            pl.BlockSpec((1, 1, 1), lambda c: (c, 0, 0)),
        ],
        out_specs=[
            pl.BlockSpec((1, _QSIZE, _DIM), lambda c: (c, 0, 0)),
            pl.BlockSpec((1, 1, _DIM), lambda c: (c, 0, 0)),
        ],
        out_shape=[
            jax.ShapeDtypeStruct((_C, _QSIZE, _DIM), f32),
            jax.ShapeDtypeStruct((_C, 1, _DIM), f32),
        ],
    )(x_sel.reshape(_B, _C, 128, _DIM), valid.reshape(_B, _C, 1, 128),
      tie2p, point_queue, cluster_center.reshape(_C, 1, _DIM),
      point_queue_ptr.astype(jnp.int32).reshape(_C, 1, 1))

    new_ptr = jnp.mod(point_queue_ptr.astype(jnp.int32) + _UPD, _QSIZE)
    return (loss[0, 0], new_queue, center.reshape(_C, _DIM), new_ptr)


# speedup vs baseline: 3.4962x; 2.2322x over previous
import jax
import jax.numpy as jnp
from jax.experimental import pallas as pl
from jax.experimental.pallas import tpu as pltpu

_C = 17
_DIM = 64
_NVIEW = 100
_QSIZE = 150
_UPD = 30
_MU = 0.99
_TEMP = 0.1
_B = 8
_N = 16384
_BC = _B * _C
_NQ = _C * _QSIZE
_NCT = _NQ + _C
_CTP = 2688
_BIG = 1 << 30


def _topk_gather_kernel(yh_ref, y_ref, tie_ref, feats_ref, x_ref, val_ref,
                        score_ref):
    f32 = jnp.float32
    yh = yh_ref[0]
    yy = y_ref[0]
    lin3 = (jax.lax.broadcasted_iota(jnp.int32, (_C, 128, 128), 1) * 128
            + jax.lax.broadcasted_iota(jnp.int32, (_C, 128, 128), 2))
    kio = jax.lax.broadcasted_iota(jnp.int32, (_C, 128), 1)
    cls3 = jax.lax.broadcasted_iota(jnp.int32, (_C, 1, 1), 0)
    mask = (yh[None] == cls3)
    sc = (mask.astype(f32) * 4.0
          + jnp.logical_and(mask, yy[None] != cls3).astype(f32)
          + tie_ref[0])
    score_ref[...] = sc
    m_g0 = jnp.max(sc, axis=2)
    l_g0 = jnp.min(jnp.where(sc == m_g0[:, :, None], lin3, _BIG), axis=2)

    def body(k, carry):
        m_g, l_g, selarr, validarr = carry
        m_c = jnp.max(m_g, axis=1, keepdims=True)
        sel = jnp.min(jnp.where(m_g == m_c, l_g, _BIG),
                      axis=1, keepdims=True)
        selarr = jnp.where(kio == k, sel, selarr)
        validarr = jnp.where(kio == k, (m_c >= 4.0).astype(f32), validarr)
        s = score_ref[...]
        s = jnp.where(lin3 == sel[:, :, None], -1.0, s)
        score_ref[...] = s
        m_g = jnp.max(s, axis=2)
        l_g = jnp.min(jnp.where(s == m_g[:, :, None], lin3, _BIG), axis=2)
        return (m_g, l_g, selarr, validarr)

    _, _, selarr, validarr = jax.lax.fori_loop(
        0, _NVIEW, body,
        (m_g0, l_g0, jnp.full((_C, 128), -1, jnp.int32),
         jnp.zeros((_C, 128), f32)))
    val_ref[0] = validarr
    ni = jax.lax.broadcasted_iota(jnp.int32, (128, _N), 1)
    feats_b = feats_ref[0]
    for c in range(_C):
        sel_col = selarr[c, :].reshape(128, 1)
        oh = (sel_col == ni).astype(f32)
        x_ref[0, c * 128:(c + 1) * 128, :] = jax.lax.dot_general(
            oh, feats_b, (((1,), (0,)), ((), ())),
            preferred_element_type=f32)


def _loss_kernel(x_ref, val_ref, ct_ref, num_ref, den_ref, loss_ref):
    step = pl.program_id(0)
    c = jnp.mod(step, _C)
    x = x_ref[0]
    nrm = jnp.sqrt(jnp.sum(x * x, axis=1, keepdims=True))
    a = x / jnp.maximum(nrm, 1e-12)
    logits = jax.lax.dot_general(a, ct_ref[...], (((1,), (0,)), ((), ())),
                                 preferred_element_type=jnp.float32) / _TEMP
    col = jax.lax.broadcasted_iota(jnp.int32, (128, _CTP), 1)
    clab = jnp.where(col < _NQ, col // _QSIZE,
                     jnp.where(col < _NCT, col - _NQ, -1))
    real = col < _NCT
    lm = jnp.where(real, logits, -1e30)
    m = jnp.max(lm, axis=1, keepdims=True)
    ex = jnp.where(real, jnp.exp(lm - m), 0.0)
    lse = jnp.log(jnp.sum(ex, axis=1, keepdims=True))
    pos = (clab == c)
    spos = jnp.sum(jnp.where(pos, lm - m - lse, 0.0), axis=1, keepdims=True)
    per = -_TEMP * (spos / (_QSIZE + 1.0))
    v = val_ref[0]
    nm = jax.lax.dot_general(v, per, (((1,), (0,)), ((), ())),
                             preferred_element_type=jnp.float32)
    dn = jnp.sum(v).reshape(1, 1)

    @pl.when(step == 0)
    def _():
        num_ref[...] = nm
        den_ref[...] = dn
        loss_ref[...] = jnp.zeros((1, 1), jnp.float32)

    @pl.when(step > 0)
    def _():
        num_ref[...] = num_ref[...] + nm
        den_ref[...] = den_ref[...] + dn

    @pl.when(step == _BC - 1)
    def _():
        loss_ref[...] = num_ref[...] / jnp.maximum(den_ref[...], 1.0)


def _queue_kernel(x_ref, val_ref, tie2_ref, q_ref, cen_ref, ptr_ref,
                  qout_ref, cout_ref):
    p0 = ptr_ref[0, 0, 0]
    val = val_ref[:, 0, 0, :]
    sv0 = val * 2.0 + tie2_ref[0]
    lin = (jax.lax.broadcasted_iota(jnp.int32, (_B, 128), 0) * 128
           + jax.lax.broadcasted_iota(jnp.int32, (_B, 128), 1))
    qout_ref[0] = q_ref[0]

    def body(k, sv):
        m = jnp.max(sv)
        sel = jnp.min(jnp.where(sv == m, lin, _BIG))
        bs = sel // 128
        js = jnp.mod(sel, 128)
        row = x_ref[pl.ds(bs, 1), 0, pl.ds(js, 1), :].reshape(1, _DIM)
        nr = jnp.sqrt(jnp.sum(row * row))
        r = row / jnp.maximum(nr, 1e-12)
        p = jnp.mod(p0 + k, _QSIZE)
        qout_ref[0, pl.ds(p, 1), :] = r
        return jnp.where(lin == sel, -20.0, sv)

    jax.lax.fori_loop(0, _UPD, body, sv0)

    x = x_ref[:, 0, :, :]
    nrm = jnp.sqrt(jnp.sum(x * x, axis=2, keepdims=True))
    an = x / jnp.maximum(nrm, 1e-12)
    wsum = jnp.sum(an * val[:, :, None], axis=(0, 1)).reshape(1, _DIM)
    cnt = jnp.maximum(jnp.sum(val), 1.0)
    newc = wsum / cnt
    newc = newc / jnp.maximum(jnp.sqrt(jnp.sum(newc * newc)), 1e-12)
    cen = _MU * cen_ref[0] + (1.0 - _MU) * newc
    cout_ref[0] = cen / jnp.maximum(jnp.sqrt(jnp.sum(cen * cen)), 1e-12)


def kernel(feats, y_hat, y, point_queue, cluster_center, point_queue_ptr):
    tie = jax.random.uniform(jax.random.key(42), (_B, _C, _N))
    tie2 = jax.random.uniform(jax.random.key(7), (_C, _B * _NVIEW))
    tie2p = jnp.pad(tie2.reshape(_C, _B, _NVIEW), ((0, 0), (0, 0), (0, 28)),
                    constant_values=-10.0)

    f32 = jnp.float32
    x_raw, valid_raw = pl.pallas_call(
        _topk_gather_kernel,
        grid=(_B,),
        in_specs=[
            pl.BlockSpec((1, 128, 128), lambda b: (b, 0, 0)),
            pl.BlockSpec((1, 128, 128), lambda b: (b, 0, 0)),
            pl.BlockSpec((1, _C, 128, 128), lambda b: (b, 0, 0, 0)),
            pl.BlockSpec((1, _N, _DIM), lambda b: (b, 0, 0)),
        ],
        out_specs=[
            pl.BlockSpec((1, _C * 128, _DIM), lambda b: (b, 0, 0)),
            pl.BlockSpec((1, _C, 128), lambda b: (b, 0, 0)),
        ],
        out_shape=[
            jax.ShapeDtypeStruct((_B, _C * 128, _DIM), f32),
            jax.ShapeDtypeStruct((_B, _C, 128), f32),
        ],
        scratch_shapes=[pltpu.VMEM((_C, 128, 128), f32)],
    )(y_hat.reshape(_B, 128, 128), y.reshape(_B, 128, 128),
      tie.reshape(_B, _C, 128, 128), feats)
    x_sel = x_raw.reshape(_BC, 128, _DIM)
    valid = valid_raw.reshape(_BC, 1, 128)

    ct = jnp.concatenate(
        [point_queue.reshape(_NQ, _DIM), cluster_center], axis=0)
    ctp = jnp.pad(ct, ((0, _CTP - _NCT), (0, 0))).T

    num, den, loss = pl.pallas_call(
        _loss_kernel,
        grid=(_BC,),
        in_specs=[
            pl.BlockSpec((1, 128, _DIM), lambda i: (i, 0, 0)),
            pl.BlockSpec((1, 1, 128), lambda i: (i, 0, 0)),
            pl.BlockSpec((_DIM, _CTP), lambda i: (0, 0)),
        ],
        out_specs=[
            pl.BlockSpec((1, 1), lambda i: (0, 0)),
            pl.BlockSpec((1, 1), lambda i: (0, 0)),
            pl.BlockSpec((1, 1), lambda i: (0, 0)),
        ],
        out_shape=[
            jax.ShapeDtypeStruct((1, 1), f32),
            jax.ShapeDtypeStruct((1, 1), f32),
            jax.ShapeDtypeStruct((1, 1), f32),
        ],
    )(x_sel, valid, ctp)

    new_queue, center = pl.pallas_call(
        _queue_kernel,
        grid=(_C,),
        in_specs=[
            pl.BlockSpec((_B, 1, 128, _DIM), lambda c: (0, c, 0, 0)),
            pl.BlockSpec((_B, 1, 1, 128), lambda c: (0, c, 0, 0)),
            pl.BlockSpec((1, _B, 128), lambda c: (c, 0, 0)),
            pl.BlockSpec((1, _QSIZE, _DIM), lambda c: (c, 0, 0)),
            pl.BlockSpec((1, 1, _DIM), lambda c: (c, 0, 0)),
            pl.BlockSpec((1, 1, 1), lambda c: (c, 0, 0)),
        ],
        out_specs=[
            pl.BlockSpec((1, _QSIZE, _DIM), lambda c: (c, 0, 0)),
            pl.BlockSpec((1, 1, _DIM), lambda c: (c, 0, 0)),
        ],
        out_shape=[
            jax.ShapeDtypeStruct((_C, _QSIZE, _DIM), f32),
            jax.ShapeDtypeStruct((_C, 1, _DIM), f32),
        ],
    )(x_sel.reshape(_B, _C, 128, _DIM), valid.reshape(_B, _C, 1, 128),
      tie2p, point_queue, cluster_center.reshape(_C, 1, _DIM),
      point_queue_ptr.astype(jnp.int32).reshape(_C, 1, 1))

    new_ptr = jnp.mod(point_queue_ptr.astype(jnp.int32) + _UPD, _QSIZE)
    return (loss[0, 0], new_queue, center.reshape(_C, _DIM), new_ptr)
